# baseline (speedup 1.0000x reference)
import jax
import jax.numpy as jnp
from jax.experimental import pallas as pl

B = 8
M = 64
G = 64
F = 192
R = B * M


def _qc_kernel(pc_ref, nm_ref, co_ref, gp_ref,
               ptw1_ref, ptb1_ref, ptw2_ref, ptb2_ref, ptw3_ref, ptb3_ref,
               nmw1_ref, nmb1_ref, nmw2_ref, nmb2_ref, nmw3_ref, nmb3_ref,
               txw1_ref, txb1_ref, txw2_ref, txb2_ref, txw3_ref, txb3_ref,
               dnw1_ref, dnb1_ref, dnw2_ref, dnb2_ref,
               clw1_ref, clb1_ref, clw2_ref, clb2_ref,
               svw1_ref, svb1_ref, svw2_ref, svb2_ref,
               probs_ref, sev_ref, proc_ref, gft_ref):
    pts = pc_ref[...].reshape(R, 3)
    nrm = nm_ref[...].reshape(R, 3)
    col = co_ref[...].reshape(R, 3)
    gp = jnp.transpose(gp_ref[...])

    def rowvec(ref):
        return ref[...].reshape(1, -1)

    def mlp(x, w1, b1, w2, b2, w3, b3):
        h = jnp.maximum(x @ w1[...] + rowvec(b1), 0.0)
        h = jnp.maximum(h @ w2[...] + rowvec(b2), 0.0)
        return h @ w3[...] + rowvec(b3)

    pf = mlp(pts, ptw1_ref, ptb1_ref, ptw2_ref, ptb2_ref, ptw3_ref, ptb3_ref)
    nf = mlp(nrm, nmw1_ref, nmb1_ref, nmw2_ref, nmb2_ref, nmw3_ref, nmb3_ref)
    tf = mlp(col, txw1_ref, txb1_ref, txw2_ref, txb2_ref, txw3_ref, txb3_ref)
    comb = jnp.concatenate([pf, nf, tf], axis=1)

    d = ((pts[:, 0:1] - gp[0:1, :]) ** 2
         + (pts[:, 1:2] - gp[1:2, :]) ** 2
         + (pts[:, 2:3] - gp[2:3, :]) ** 2)
    minv = jnp.min(d, axis=1, keepdims=True)
    gio = jax.lax.broadcasted_iota(jnp.int32, (R, G), 1)
    idxc = jnp.min(jnp.where(d == minv, gio, G), axis=1, keepdims=True)

    rio_col = jax.lax.broadcasted_iota(jnp.int32, (R, 1), 0)
    idx_glob = idxc + (rio_col // M) * G
    gio_full = jax.lax.broadcasted_iota(jnp.int32, (R, R), 1)
    rio_full = jax.lax.broadcasted_iota(jnp.int32, (R, R), 0)
    onehot = idx_glob == gio_full
    val = jnp.where(onehot, rio_full + 1, 0)
    wins = jnp.max(val, axis=0, keepdims=True)
    selT = ((val == wins) & (wins > 0)).astype(jnp.float32)
    gf = jax.lax.dot_general(
        selT, comb, (((0,), (0,)), ((), ())),
        preferred_element_type=jnp.float32)

    hd = jnp.maximum(gf @ dnw1_ref[...] + rowvec(dnb1_ref), 0.0)
    defect = hd @ dnw2_ref[...] + rowvec(dnb2_ref)

    gft_ref[...] = jnp.transpose(gf.reshape(B, G, F), (0, 2, 1))
    proc_ref[...] = jnp.transpose(defect.reshape(B, G, 64), (0, 2, 1))

    hc = jnp.maximum(defect @ clw1_ref[...] + rowvec(clb1_ref), 0.0)
    logits = hc @ clw2_ref[...] + rowvec(clb2_ref)
    probs_ref[...] = jax.nn.softmax(logits, axis=-1).reshape(B, G, 5)

    hs = jnp.maximum(defect @ svw1_ref[...] + rowvec(svb1_ref), 0.0)
    sev_ref[...] = jax.nn.sigmoid(hs @ svw2_ref[...] + rowvec(svb2_ref)).reshape(B, G, 1)


def kernel(point_cloud, normals, colors, grid_points, params):
    N = point_cloud.shape[1]
    first64 = pl.BlockSpec((B, M, 3), lambda i: (0, 0, 0))
    full = lambda a: pl.BlockSpec(a.shape, lambda i: (0,) * a.ndim)

    p = params
    weight_args = [
        p["pt_W1"], p["pt_b1"], p["pt_W2"], p["pt_b2"], p["pt_W3"], p["pt_b3"],
        p["nm_W1"], p["nm_b1"], p["nm_W2"], p["nm_b2"], p["nm_W3"], p["nm_b3"],
        p["tx_W1"], p["tx_b1"], p["tx_W2"], p["tx_b2"], p["tx_W3"], p["tx_b3"],
        p["dn_W1"], p["dn_b1"], p["dn_W2"], p["dn_b2"],
        p["cl_W1"], p["cl_b1"], p["cl_W2"], p["cl_b2"],
        p["sv_W1"], p["sv_b1"], p["sv_W2"], p["sv_b2"],
    ]

    out_shapes = (
        jax.ShapeDtypeStruct((B, G, 5), jnp.float32),
        jax.ShapeDtypeStruct((B, G, 1), jnp.float32),
        jax.ShapeDtypeStruct((B, 64, G), jnp.float32),
        jax.ShapeDtypeStruct((B, F, G), jnp.float32),
    )

    probs, sev, proc, gft = pl.pallas_call(
        _qc_kernel,
        out_shape=out_shapes,
        grid=(1,),
        in_specs=[first64, first64, first64, full(grid_points)]
        + [full(w) for w in weight_args],
        out_specs=tuple(
            pl.BlockSpec(s.shape, lambda i: (0,) * len(s.shape))
            for s in out_shapes),
    )(point_cloud, normals, colors, grid_points, *weight_args)

    return probs, sev[..., 0], proc, gft


# speedup vs baseline: 1.6321x; 1.6321x over previous
import jax
import jax.numpy as jnp
from jax.experimental import pallas as pl

B = 8
M = 64
G = 64
F = 192
R = B * M

_PARAM_SHAPES = [
    ("pt_W1", 3, 16), ("pt_b1", 1, 16), ("pt_W2", 16, 32), ("pt_b2", 1, 32),
    ("pt_W3", 32, 64), ("pt_b3", 1, 64),
    ("nm_W1", 3, 16), ("nm_b1", 1, 16), ("nm_W2", 16, 32), ("nm_b2", 1, 32),
    ("nm_W3", 32, 64), ("nm_b3", 1, 64),
    ("tx_W1", 3, 16), ("tx_b1", 1, 16), ("tx_W2", 16, 32), ("tx_b2", 1, 32),
    ("tx_W3", 32, 64), ("tx_b3", 1, 64),
    ("dn_W1", 192, 64), ("dn_b1", 1, 64), ("dn_W2", 64, 64), ("dn_b2", 1, 64),
    ("cl_W1", 64, 32), ("cl_b1", 1, 32), ("cl_W2", 32, 5), ("cl_b2", 1, 5),
    ("sv_W1", 64, 32), ("sv_b1", 1, 32), ("sv_W2", 32, 1), ("sv_b2", 1, 1),
]
_OFFSETS = {}
_rows = 0
for _n, _r, _c in _PARAM_SHAPES:
    _OFFSETS[_n] = _rows
    _rows += _r
_TOTAL_ROWS = _rows


def _qc_kernel(pc_ref, nm_ref, co_ref, gp_ref, pk_ref,
               probs_ref, sev_ref, proc_ref, gft_ref):
    pts = pc_ref[...].reshape(R, 3)
    nrm = nm_ref[...].reshape(R, 3)
    col = co_ref[...].reshape(R, 3)
    gp = jnp.transpose(gp_ref[...])
    pk = pk_ref[...]

    def w(name, r, c):
        o = _OFFSETS[name]
        return pk[o:o + r, :c]

    def mlp(x, pre):
        h = jnp.maximum(x @ w(pre + "_W1", 3, 16) + w(pre + "_b1", 1, 16), 0.0)
        h = jnp.maximum(h @ w(pre + "_W2", 16, 32) + w(pre + "_b2", 1, 32), 0.0)
        return h @ w(pre + "_W3", 32, 64) + w(pre + "_b3", 1, 64)

    comb = jnp.concatenate(
        [mlp(pts, "pt"), mlp(nrm, "nm"), mlp(col, "tx")], axis=1)

    d = ((pts[:, 0:1] - gp[0:1, :]) ** 2
         + (pts[:, 1:2] - gp[1:2, :]) ** 2
         + (pts[:, 2:3] - gp[2:3, :]) ** 2)
    minv = jnp.min(d, axis=1, keepdims=True)
    gio = jax.lax.broadcasted_iota(jnp.int32, (R, G), 1)
    idxc = jnp.min(jnp.where(d == minv, gio, G), axis=1, keepdims=True)

    rio_col = jax.lax.broadcasted_iota(jnp.int32, (R, 1), 0)
    idx_glob = idxc + (rio_col // M) * G
    gio_full = jax.lax.broadcasted_iota(jnp.int32, (R, R), 1)
    rio_full = jax.lax.broadcasted_iota(jnp.int32, (R, R), 0)
    onehot = idx_glob == gio_full
    val = jnp.where(onehot, rio_full + 1, 0)
    wins = jnp.max(val, axis=0, keepdims=True)
    selT = ((val == wins) & (wins > 0)).astype(jnp.float32)
    gf = jax.lax.dot_general(
        selT, comb, (((0,), (0,)), ((), ())),
        preferred_element_type=jnp.float32)

    hd = jnp.maximum(gf @ w("dn_W1", 192, 64) + w("dn_b1", 1, 64), 0.0)
    defect = hd @ w("dn_W2", 64, 64) + w("dn_b2", 1, 64)

    gft_ref[...] = jnp.transpose(gf.reshape(B, G, F), (0, 2, 1))
    proc_ref[...] = jnp.transpose(defect.reshape(B, G, 64), (0, 2, 1))

    hc = jnp.maximum(defect @ w("cl_W1", 64, 32) + w("cl_b1", 1, 32), 0.0)
    logits = hc @ w("cl_W2", 32, 5) + w("cl_b2", 1, 5)
    probs_ref[...] = jax.nn.softmax(logits, axis=-1).reshape(B, G, 5)

    hs = jnp.maximum(defect @ w("sv_W1", 64, 32) + w("sv_b1", 1, 32), 0.0)
    sev_ref[...] = jax.nn.sigmoid(
        hs @ w("sv_W2", 32, 1) + w("sv_b2", 1, 1)).reshape(B, G, 1)


def kernel(point_cloud, normals, colors, grid_points, params):
    pts64 = point_cloud[:, :M]
    nrm64 = normals[:, :M]
    col64 = colors[:, :M]

    blocks = []
    for name, r, c in _PARAM_SHAPES:
        v = params[name].reshape(r, c)
        blocks.append(jnp.pad(v, ((0, 0), (0, 128 - c))))
    packed = jnp.concatenate(blocks, axis=0)

    out_shapes = (
        jax.ShapeDtypeStruct((B, G, 5), jnp.float32),
        jax.ShapeDtypeStruct((B, G, 1), jnp.float32),
        jax.ShapeDtypeStruct((B, 64, G), jnp.float32),
        jax.ShapeDtypeStruct((B, F, G), jnp.float32),
    )

    probs, sev, proc, gft = pl.pallas_call(
        _qc_kernel,
        out_shape=out_shapes,
    )(pts64, nrm64, col64, grid_points, packed)

    return probs, sev[..., 0], proc, gft


# speedup vs baseline: 4.5895x; 2.8121x over previous
import jax
import jax.numpy as jnp
from jax.experimental import pallas as pl

B = 8
M = 64
G = 64
F = 192
R = B * M


def _qc_kernel(pc_ref, nm_ref, co_ref, gp_ref,
               ptw1_ref, ptb1_ref, ptw2_ref, ptb2_ref, ptw3_ref, ptb3_ref,
               nmw1_ref, nmb1_ref, nmw2_ref, nmb2_ref, nmw3_ref, nmb3_ref,
               txw1_ref, txb1_ref, txw2_ref, txb2_ref, txw3_ref, txb3_ref,
               dnw1_ref, dnb1_ref, dnw2_ref, dnb2_ref,
               clw1_ref, clb1_ref, clw2_ref, clb2_ref,
               svw1_ref, svb1_ref, svw2_ref, svb2_ref,
               probs_ref, sev_ref, proc_ref, gft_ref):
    pts = pc_ref[...].reshape(R, 3)
    nrm = nm_ref[...].reshape(R, 3)
    col = co_ref[...].reshape(R, 3)
    gp = jnp.transpose(gp_ref[...])

    def rowvec(ref):
        return ref[...].reshape(1, -1)

    def mlp(x, w1, b1, w2, b2, w3, b3):
        h = jnp.maximum(x @ w1[...] + rowvec(b1), 0.0)
        h = jnp.maximum(h @ w2[...] + rowvec(b2), 0.0)
        return h @ w3[...] + rowvec(b3)

    pf = mlp(pts, ptw1_ref, ptb1_ref, ptw2_ref, ptb2_ref, ptw3_ref, ptb3_ref)
    nf = mlp(nrm, nmw1_ref, nmb1_ref, nmw2_ref, nmb2_ref, nmw3_ref, nmb3_ref)
    tf = mlp(col, txw1_ref, txb1_ref, txw2_ref, txb2_ref, txw3_ref, txb3_ref)
    comb = jnp.concatenate([pf, nf, tf], axis=1)

    d = ((pts[:, 0:1] - gp[0:1, :]) ** 2
         + (pts[:, 1:2] - gp[1:2, :]) ** 2
         + (pts[:, 2:3] - gp[2:3, :]) ** 2)
    minv = jnp.min(d, axis=1, keepdims=True)
    gio = jax.lax.broadcasted_iota(jnp.int32, (R, G), 1)
    idxc = jnp.min(jnp.where(d == minv, gio, G), axis=1, keepdims=True)

    rio_col = jax.lax.broadcasted_iota(jnp.int32, (R, 1), 0)
    idx_glob = idxc + (rio_col // M) * G
    gio_full = jax.lax.broadcasted_iota(jnp.int32, (R, R), 1)
    rio_full = jax.lax.broadcasted_iota(jnp.int32, (R, R), 0)
    onehot = idx_glob == gio_full
    val = jnp.where(onehot, rio_full + 1, 0)
    wins = jnp.max(val, axis=0, keepdims=True)
    selT = ((val == wins) & (wins > 0)).astype(jnp.float32)
    gf = jax.lax.dot_general(
        selT, comb, (((0,), (0,)), ((), ())),
        preferred_element_type=jnp.float32)

    hd = jnp.maximum(gf @ dnw1_ref[...] + rowvec(dnb1_ref), 0.0)
    defect = hd @ dnw2_ref[...] + rowvec(dnb2_ref)

    gft_ref[...] = jnp.transpose(gf.reshape(B, G, F), (0, 2, 1))
    proc_ref[...] = jnp.transpose(defect.reshape(B, G, 64), (0, 2, 1))

    hc = jnp.maximum(defect @ clw1_ref[...] + rowvec(clb1_ref), 0.0)
    logits = hc @ clw2_ref[...] + rowvec(clb2_ref)
    probs_ref[...] = jax.nn.softmax(logits, axis=-1).reshape(B, G, 5)

    hs = jnp.maximum(defect @ svw1_ref[...] + rowvec(svb1_ref), 0.0)
    sev_ref[...] = jax.nn.sigmoid(
        hs @ svw2_ref[...] + rowvec(svb2_ref)).reshape(B, G, 1)


def kernel(point_cloud, normals, colors, grid_points, params):
    pts64 = point_cloud[:, :M]
    nrm64 = normals[:, :M]
    col64 = colors[:, :M]

    p = params
    weight_args = [
        p["pt_W1"], p["pt_b1"], p["pt_W2"], p["pt_b2"], p["pt_W3"], p["pt_b3"],
        p["nm_W1"], p["nm_b1"], p["nm_W2"], p["nm_b2"], p["nm_W3"], p["nm_b3"],
        p["tx_W1"], p["tx_b1"], p["tx_W2"], p["tx_b2"], p["tx_W3"], p["tx_b3"],
        p["dn_W1"], p["dn_b1"], p["dn_W2"], p["dn_b2"],
        p["cl_W1"], p["cl_b1"], p["cl_W2"], p["cl_b2"],
        p["sv_W1"], p["sv_b1"], p["sv_W2"], p["sv_b2"],
    ]

    out_shapes = (
        jax.ShapeDtypeStruct((B, G, 5), jnp.float32),
        jax.ShapeDtypeStruct((B, G, 1), jnp.float32),
        jax.ShapeDtypeStruct((B, 64, G), jnp.float32),
        jax.ShapeDtypeStruct((B, F, G), jnp.float32),
    )

    probs, sev, proc, gft = pl.pallas_call(
        _qc_kernel,
        out_shape=out_shapes,
    )(pts64, nrm64, col64, grid_points, *weight_args)

    return probs, sev[..., 0], proc, gft


# speedup vs baseline: 9.1858x; 2.0015x over previous
import jax
import jax.numpy as jnp
from jax.experimental import pallas as pl

B = 8
M = 64
G = 64
F = 192
R = B * M

_DOT_RT = (((1,), (1,)), ((), ()))


def _qc_kernel(pc_ref, nm_ref, co_ref, gpt_ref,
               ptw1_ref, ptb1_ref, ptw2_ref, ptb2_ref, ptw3_ref, ptb3_ref,
               nmw1_ref, nmb1_ref, nmw2_ref, nmb2_ref, nmw3_ref, nmb3_ref,
               txw1_ref, txb1_ref, txw2_ref, txb2_ref, txw3_ref, txb3_ref,
               dnw1t_ref, dnb1_ref, dnw2_ref, dnb2_ref,
               clw1t_ref, clb1_ref, clw2t_ref, clb2_ref,
               svw1t_ref, svb1_ref, svw2t_ref, svb2_ref,
               probs_ref, sev_ref, proc_ref, gf_ref):
    pts = pc_ref[...].reshape(R, 3)
    nrm = nm_ref[...].reshape(R, 3)
    col = co_ref[...].reshape(R, 3)
    gp = gpt_ref[...]

    def rowvec(ref):
        return ref[...].reshape(1, -1)

    def matT(x, wt_ref):
        return jax.lax.dot_general(x, wt_ref[...], _DOT_RT,
                                   preferred_element_type=jnp.float32)

    def mlp(x, w1, b1, w2, b2, w3, b3):
        h = jnp.maximum(x @ w1[...] + rowvec(b1), 0.0)
        h = jnp.maximum(h @ w2[...] + rowvec(b2), 0.0)
        return h @ w3[...] + rowvec(b3)

    pf = mlp(pts, ptw1_ref, ptb1_ref, ptw2_ref, ptb2_ref, ptw3_ref, ptb3_ref)
    nf = mlp(nrm, nmw1_ref, nmb1_ref, nmw2_ref, nmb2_ref, nmw3_ref, nmb3_ref)
    tf = mlp(col, txw1_ref, txb1_ref, txw2_ref, txb2_ref, txw3_ref, txb3_ref)
    comb = jnp.concatenate([pf, nf, tf], axis=1)

    d = ((pts[:, 0:1] - gp[0:1, :]) ** 2
         + (pts[:, 1:2] - gp[1:2, :]) ** 2
         + (pts[:, 2:3] - gp[2:3, :]) ** 2)
    minv = jnp.min(d, axis=1, keepdims=True)
    gio = jax.lax.broadcasted_iota(jnp.int32, (R, G), 1)
    idxc = jnp.min(jnp.where(d == minv, gio, G), axis=1, keepdims=True)

    rio_col = jax.lax.broadcasted_iota(jnp.int32, (R, 1), 0)
    idx_glob = idxc + (rio_col // M) * G
    gio_full = jax.lax.broadcasted_iota(jnp.int32, (R, R), 1)
    rio_full = jax.lax.broadcasted_iota(jnp.int32, (R, R), 0)
    onehot = idx_glob == gio_full
    val = jnp.where(onehot, rio_full + 1, 0)
    wins = jnp.max(val, axis=0, keepdims=True)
    selT = ((val == wins) & (wins > 0)).astype(jnp.float32)
    gf = jax.lax.dot_general(
        selT, comb, (((0,), (0,)), ((), ())),
        preferred_element_type=jnp.float32)

    gf_ref[...] = gf.reshape(B, G, F)

    hd = jnp.maximum(matT(gf, dnw1t_ref) + rowvec(dnb1_ref), 0.0)
    defect = hd @ dnw2_ref[...] + rowvec(dnb2_ref)

    proc_ref[...] = jnp.transpose(defect.reshape(B, G, 64), (0, 2, 1))

    hc = jnp.maximum(matT(defect, clw1t_ref) + rowvec(clb1_ref), 0.0)
    logits = matT(hc, clw2t_ref) + rowvec(clb2_ref)
    probs_ref[...] = jax.nn.softmax(logits, axis=-1).reshape(B, G, 5)

    hs = jnp.maximum(matT(defect, svw1t_ref) + rowvec(svb1_ref), 0.0)
    sev_pre = jnp.sum(hs * rowvec(svw2t_ref), axis=1, keepdims=True)
    sev = jax.nn.sigmoid(sev_pre + rowvec(svb2_ref))
    sevT = jnp.transpose(sev)
    for b in range(B):
        sev_ref[b:b + 1, :] = sevT[:, b * M:(b + 1) * M]


def kernel(point_cloud, normals, colors, grid_points, params):
    pts64 = point_cloud[:, :M]
    nrm64 = normals[:, :M]
    col64 = colors[:, :M]

    p = params
    t = jnp.transpose
    weight_args = [
        p["pt_W1"], p["pt_b1"], p["pt_W2"], p["pt_b2"], p["pt_W3"], p["pt_b3"],
        p["nm_W1"], p["nm_b1"], p["nm_W2"], p["nm_b2"], p["nm_W3"], p["nm_b3"],
        p["tx_W1"], p["tx_b1"], p["tx_W2"], p["tx_b2"], p["tx_W3"], p["tx_b3"],
        t(p["dn_W1"]), p["dn_b1"], p["dn_W2"], p["dn_b2"],
        t(p["cl_W1"]), p["cl_b1"], t(p["cl_W2"]), p["cl_b2"],
        t(p["sv_W1"]), p["sv_b1"], t(p["sv_W2"]), p["sv_b2"],
    ]

    out_shapes = (
        jax.ShapeDtypeStruct((B, G, 5), jnp.float32),
        jax.ShapeDtypeStruct((B, G), jnp.float32),
        jax.ShapeDtypeStruct((B, 64, G), jnp.float32),
        jax.ShapeDtypeStruct((B, G, F), jnp.float32),
    )

    probs, sev, proc, gf = pl.pallas_call(
        _qc_kernel,
        out_shape=out_shapes,
    )(pts64, nrm64, col64, t(grid_points), *weight_args)

    return probs, sev, proc, jnp.transpose(gf, (0, 2, 1))


# speedup vs baseline: 11.8676x; 1.2919x over previous
import jax
import jax.numpy as jnp
from jax.experimental import pallas as pl

B = 8
M = 64
G = 64
F = 192
R = B * M

_DOT_RT = (((1,), (1,)), ((), ()))
_DOT_LT = (((0,), (0,)), ((), ()))


def _qc_kernel(pc_ref, nm_ref, co_ref, gpt_ref,
               ptw1_ref, ptb1_ref, ptw2_ref, ptb2_ref, ptw3_ref, ptb3_ref,
               nmw1_ref, nmb1_ref, nmw2_ref, nmb2_ref, nmw3_ref, nmb3_ref,
               txw1_ref, txb1_ref, txw2_ref, txb2_ref, txw3_ref, txb3_ref,
               dnw1t_ref, dnb1_ref, dnw2_ref, dnb2_ref,
               clw1t_ref, clb1_ref, clw2t_ref, clb2_ref,
               svw1t_ref, svb1_ref, svw2t_ref, svb2_ref,
               probs_ref, sev_ref, proc_ref, gf_ref):
    def coords(ref):
        x24 = ref[...].reshape(3 * B, M)
        rows = [
            jnp.concatenate([x24[k * B + b:k * B + b + 1, :]
                             for b in range(B)], axis=1)
            for k in range(3)
        ]
        return jnp.concatenate(rows, axis=0)

    pts3 = coords(pc_ref)
    nrm3 = coords(nm_ref)
    col3 = coords(co_ref)
    gp = jnp.transpose(gpt_ref[...])

    def rowvec(ref):
        return ref[...].reshape(1, -1)

    def matT(x, wt_ref):
        return jax.lax.dot_general(x, wt_ref[...], _DOT_RT,
                                   preferred_element_type=jnp.float32)

    def mlp(x3, w1, b1, w2, b2, w3, b3):
        h = jax.lax.dot_general(x3, w1[...], _DOT_LT,
                                preferred_element_type=jnp.float32)
        h = jnp.maximum(h + rowvec(b1), 0.0)
        h = jnp.maximum(h @ w2[...] + rowvec(b2), 0.0)
        return h @ w3[...] + rowvec(b3)

    pf = mlp(pts3, ptw1_ref, ptb1_ref, ptw2_ref, ptb2_ref, ptw3_ref, ptb3_ref)
    nf = mlp(nrm3, nmw1_ref, nmb1_ref, nmw2_ref, nmb2_ref, nmw3_ref, nmb3_ref)
    tf = mlp(col3, txw1_ref, txb1_ref, txw2_ref, txb2_ref, txw3_ref, txb3_ref)
    comb = jnp.concatenate([pf, nf, tf], axis=1)

    dT = ((gp[:, 0:1] - pts3[0:1, :]) ** 2
          + (gp[:, 1:2] - pts3[1:2, :]) ** 2
          + (gp[:, 2:3] - pts3[2:3, :]) ** 2)
    minv = jnp.min(dT, axis=0, keepdims=True)
    gio_s = jax.lax.broadcasted_iota(jnp.int32, (G, R), 0)
    idx = jnp.min(jnp.where(dT == minv, gio_s, G), axis=0, keepdims=True)

    cell_col = jax.lax.broadcasted_iota(jnp.int32, (G, M), 0)
    lane_io = jax.lax.broadcasted_iota(jnp.int32, (G, M), 1)

    gfs = []
    for b in range(B):
        idx_b = idx[:, b * M:(b + 1) * M]
        onehot = cell_col == idx_b
        val = jnp.where(onehot, lane_io + 1, 0)
        wins = jnp.max(val, axis=1, keepdims=True)
        sel = ((val == wins) & (wins > 0)).astype(jnp.float32)
        gf_b = sel @ comb[b * M:(b + 1) * M]
        gf_ref[b, :, :] = gf_b
        gfs.append(gf_b)
    gf = jnp.concatenate(gfs, axis=0)

    hd = jnp.maximum(matT(gf, dnw1t_ref) + rowvec(dnb1_ref), 0.0)
    defect = hd @ dnw2_ref[...] + rowvec(dnb2_ref)

    proc_ref[...] = jnp.transpose(defect.reshape(B, G, 64), (0, 2, 1))

    hc = jnp.maximum(matT(defect, clw1t_ref) + rowvec(clb1_ref), 0.0)
    logits = matT(hc, clw2t_ref) + rowvec(clb2_ref)
    probsT = jnp.transpose(jax.nn.softmax(logits, axis=-1))
    for b in range(B):
        probs_ref[:, b, :] = probsT[:, b * M:(b + 1) * M]

    hs = jnp.maximum(matT(defect, svw1t_ref) + rowvec(svb1_ref), 0.0)
    sev_pre = jnp.sum(hs * rowvec(svw2t_ref), axis=1, keepdims=True)
    sevT = jnp.transpose(jax.nn.sigmoid(sev_pre + svb2_ref[...]))
    for b in range(B):
        sev_ref[b:b + 1, :] = sevT[:, b * M:(b + 1) * M]


def kernel(point_cloud, normals, colors, grid_points, params):
    t = jnp.transpose
    pts_t = t(point_cloud, (2, 0, 1))[:, :, :M]
    nrm_t = t(normals, (2, 0, 1))[:, :, :M]
    col_t = t(colors, (2, 0, 1))[:, :, :M]

    p = params
    weight_args = [
        p["pt_W1"], p["pt_b1"], p["pt_W2"], p["pt_b2"], p["pt_W3"], p["pt_b3"],
        p["nm_W1"], p["nm_b1"], p["nm_W2"], p["nm_b2"], p["nm_W3"], p["nm_b3"],
        p["tx_W1"], p["tx_b1"], p["tx_W2"], p["tx_b2"], p["tx_W3"], p["tx_b3"],
        t(p["dn_W1"]), p["dn_b1"], p["dn_W2"], p["dn_b2"],
        t(p["cl_W1"]), p["cl_b1"], t(p["cl_W2"]), p["cl_b2"],
        t(p["sv_W1"]), p["sv_b1"], t(p["sv_W2"]), p["sv_b2"].reshape(1, 1),
    ]

    out_shapes = (
        jax.ShapeDtypeStruct((5, B, G), jnp.float32),
        jax.ShapeDtypeStruct((B, G), jnp.float32),
        jax.ShapeDtypeStruct((B, 64, G), jnp.float32),
        jax.ShapeDtypeStruct((B, G, F), jnp.float32),
    )

    probs5, sev, proc, gf = pl.pallas_call(
        _qc_kernel,
        out_shape=out_shapes,
    )(pts_t, nrm_t, col_t, t(grid_points), *weight_args)

    return (jnp.transpose(probs5, (1, 2, 0)), sev, proc,
            jnp.transpose(gf, (0, 2, 1)))


# speedup vs baseline: 20.3789x; 1.7172x over previous
import jax
import jax.numpy as jnp
from jax.experimental import pallas as pl

B = 8
M = 64
G = 64
F = 192
R = B * M

_DOT_RT = (((1,), (1,)), ((), ()))
_DOT_LT = (((0,), (0,)), ((), ()))


def _qc_kernel(pc_ref, nm_ref, co_ref, gpt_ref,
               ptw1_ref, ptb1_ref, ptw2_ref, ptb2_ref, ptw3_ref, ptb3_ref,
               nmw1_ref, nmb1_ref, nmw2_ref, nmb2_ref, nmw3_ref, nmb3_ref,
               txw1_ref, txb1_ref, txw2_ref, txb2_ref, txw3_ref, txb3_ref,
               dnw1t_ref, dnb1_ref, dnw2_ref, dnb2_ref,
               clw1t_ref, clb1_ref, clw2t_ref, clb2_ref,
               svw1t_ref, svb1_ref, svw2t_ref, svb2_ref,
               probs_ref, sev_ref, proc_ref, gf_ref):
    def coords(ref):
        x24 = ref[...].reshape(3 * B, 128)
        rows = [
            jnp.concatenate([x24[k * B + b:k * B + b + 1, :M]
                             for b in range(B)], axis=1)
            for k in range(3)
        ]
        return jnp.concatenate(rows, axis=0)

    pts3 = coords(pc_ref)
    nrm3 = coords(nm_ref)
    col3 = coords(co_ref)
    gp = jnp.transpose(gpt_ref[...])

    def rowvec(ref):
        return ref[...].reshape(1, -1)

    def matT(x, wt_ref):
        return jax.lax.dot_general(x, wt_ref[...], _DOT_RT,
                                   preferred_element_type=jnp.float32)

    def mlp(x3, w1, b1, w2, b2, w3, b3):
        h = jax.lax.dot_general(x3, w1[...], _DOT_LT,
                                preferred_element_type=jnp.float32)
        h = jnp.maximum(h + rowvec(b1), 0.0)
        h = jnp.maximum(h @ w2[...] + rowvec(b2), 0.0)
        return h @ w3[...] + rowvec(b3)

    pf = mlp(pts3, ptw1_ref, ptb1_ref, ptw2_ref, ptb2_ref, ptw3_ref, ptb3_ref)
    nf = mlp(nrm3, nmw1_ref, nmb1_ref, nmw2_ref, nmb2_ref, nmw3_ref, nmb3_ref)
    tf = mlp(col3, txw1_ref, txb1_ref, txw2_ref, txb2_ref, txw3_ref, txb3_ref)
    comb = jnp.concatenate([pf, nf, tf], axis=1)

    dT = ((gp[:, 0:1] - pts3[0:1, :]) ** 2
          + (gp[:, 1:2] - pts3[1:2, :]) ** 2
          + (gp[:, 2:3] - pts3[2:3, :]) ** 2)
    minv = jnp.min(dT, axis=0, keepdims=True)
    gio_s = jax.lax.broadcasted_iota(jnp.int32, (G, R), 0)
    idx = jnp.min(jnp.where(dT == minv, gio_s, G), axis=0, keepdims=True)

    cell_col = jax.lax.broadcasted_iota(jnp.int32, (G, M), 0)
    lane_io = jax.lax.broadcasted_iota(jnp.int32, (G, M), 1)

    gfs = []
    for b in range(B):
        idx_b = idx[:, b * M:(b + 1) * M]
        onehot = cell_col == idx_b
        val = jnp.where(onehot, lane_io + 1, 0)
        wins = jnp.max(val, axis=1, keepdims=True)
        sel = ((val == wins) & (wins > 0)).astype(jnp.float32)
        gf_b = sel @ comb[b * M:(b + 1) * M]
        gf_ref[b, :, :] = gf_b
        gfs.append(gf_b)
    gf = jnp.concatenate(gfs, axis=0)

    hd = jnp.maximum(matT(gf, dnw1t_ref) + rowvec(dnb1_ref), 0.0)
    defect = hd @ dnw2_ref[...] + rowvec(dnb2_ref)

    proc_ref[...] = jnp.transpose(defect.reshape(B, G, 64), (0, 2, 1))

    hc = jnp.maximum(matT(defect, clw1t_ref) + rowvec(clb1_ref), 0.0)
    logits = matT(hc, clw2t_ref) + rowvec(clb2_ref)
    probsT = jnp.transpose(jax.nn.softmax(logits, axis=-1))
    for b in range(B):
        probs_ref[:, b, :] = probsT[:, b * M:(b + 1) * M]

    hs = jnp.maximum(matT(defect, svw1t_ref) + rowvec(svb1_ref), 0.0)
    sev_pre = jnp.sum(hs * rowvec(svw2t_ref), axis=1, keepdims=True)
    sevT = jnp.transpose(jax.nn.sigmoid(sev_pre + svb2_ref[...]))
    for b in range(B):
        sev_ref[b:b + 1, :] = sevT[:, b * M:(b + 1) * M]


def kernel(point_cloud, normals, colors, grid_points, params):
    t = jnp.transpose
    pts_t = t(point_cloud, (2, 0, 1))
    nrm_t = t(normals, (2, 0, 1))
    col_t = t(colors, (2, 0, 1))

    p = params
    weight_args = [
        p["pt_W1"], p["pt_b1"], p["pt_W2"], p["pt_b2"], p["pt_W3"], p["pt_b3"],
        p["nm_W1"], p["nm_b1"], p["nm_W2"], p["nm_b2"], p["nm_W3"], p["nm_b3"],
        p["tx_W1"], p["tx_b1"], p["tx_W2"], p["tx_b2"], p["tx_W3"], p["tx_b3"],
        t(p["dn_W1"]), p["dn_b1"], p["dn_W2"], p["dn_b2"],
        t(p["cl_W1"]), p["cl_b1"], t(p["cl_W2"]), p["cl_b2"],
        t(p["sv_W1"]), p["sv_b1"], t(p["sv_W2"]), p["sv_b2"].reshape(1, 1),
    ]

    out_shapes = (
        jax.ShapeDtypeStruct((5, B, G), jnp.float32),
        jax.ShapeDtypeStruct((B, G), jnp.float32),
        jax.ShapeDtypeStruct((B, 64, G), jnp.float32),
        jax.ShapeDtypeStruct((B, G, F), jnp.float32),
    )

    first64 = pl.BlockSpec((3, B, 128), lambda i: (0, 0, 0))
    full = lambda a: pl.BlockSpec(a.shape, lambda i: (0,) * a.ndim)
    gpt = t(grid_points)

    probs5, sev, proc, gf = pl.pallas_call(
        _qc_kernel,
        out_shape=out_shapes,
        grid=(1,),
        in_specs=[first64, first64, first64, full(gpt)]
        + [full(w) for w in weight_args],
        out_specs=tuple(
            pl.BlockSpec(s.shape, lambda i, n=len(s.shape): (0,) * n)
            for s in out_shapes),
    )(pts_t, nrm_t, col_t, gpt, *weight_args)

    return (jnp.transpose(probs5, (1, 2, 0)), sev, proc,
            jnp.transpose(gf, (0, 2, 1)))


# speedup vs baseline: 20.7505x; 1.0182x over previous
import jax
import jax.numpy as jnp
from jax.experimental import pallas as pl

B = 8
M = 64
G = 64
F = 192
R = B * M

_DOT_RT = (((1,), (1,)), ((), ()))
_DOT_LT = (((0,), (0,)), ((), ()))


def _qc_kernel(pc_ref, nm_ref, co_ref, gpt_ref,
               ptw1_ref, ptb1_ref, ptw2_ref, ptb2_ref, ptw3_ref, ptb3_ref,
               nmw1_ref, nmb1_ref, nmw2_ref, nmb2_ref, nmw3_ref, nmb3_ref,
               txw1_ref, txb1_ref, txw2_ref, txb2_ref, txw3_ref, txb3_ref,
               dnw1t_ref, dnb1_ref, dnw2_ref, dnb2_ref,
               clw1t_ref, clb1_ref, clw2t_ref, clb2_ref,
               svw1t_ref, svb1_ref, svw2t_ref, svb2_ref,
               probs_ref, sev_ref, proc_ref, gf_ref):
    def coords(ref):
        x24 = ref[...].reshape(3 * B, 128)
        rows = [
            jnp.concatenate([x24[k * B + b:k * B + b + 1, :M]
                             for b in range(B)], axis=1)
            for k in range(3)
        ]
        return jnp.concatenate(rows, axis=0)

    pts3 = coords(pc_ref)
    nrm3 = coords(nm_ref)
    col3 = coords(co_ref)
    gp = jnp.transpose(gpt_ref[...])

    def rowvec(ref):
        return ref[...].reshape(1, -1)

    def matT(x, wt_ref):
        return jax.lax.dot_general(x, wt_ref[...], _DOT_RT,
                                   preferred_element_type=jnp.float32)

    def mlp(x3, w1, b1, w2, b2, w3, b3):
        h = jax.lax.dot_general(x3, w1[...], _DOT_LT,
                                preferred_element_type=jnp.float32)
        h = jnp.maximum(h + rowvec(b1), 0.0)
        h = jnp.maximum(h @ w2[...] + rowvec(b2), 0.0)
        return h @ w3[...] + rowvec(b3)

    pf = mlp(pts3, ptw1_ref, ptb1_ref, ptw2_ref, ptb2_ref, ptw3_ref, ptb3_ref)
    nf = mlp(nrm3, nmw1_ref, nmb1_ref, nmw2_ref, nmb2_ref, nmw3_ref, nmb3_ref)
    tf = mlp(col3, txw1_ref, txb1_ref, txw2_ref, txb2_ref, txw3_ref, txb3_ref)
    comb = jnp.concatenate([pf, nf, tf], axis=1)

    dT = ((gp[:, 0:1] - pts3[0:1, :]) ** 2
          + (gp[:, 1:2] - pts3[1:2, :]) ** 2
          + (gp[:, 2:3] - pts3[2:3, :]) ** 2)
    minv = jnp.min(dT, axis=0, keepdims=True)
    gio_s = jax.lax.broadcasted_iota(jnp.int32, (G, R), 0)
    idx = jnp.min(jnp.where(dT == minv, gio_s, G), axis=0, keepdims=True)

    cell_col = jax.lax.broadcasted_iota(jnp.int32, (G, M), 0)
    lane_io = jax.lax.broadcasted_iota(jnp.int32, (G, M), 1)

    gfs = []
    for b in range(B):
        idx_b = idx[:, b * M:(b + 1) * M]
        onehot = cell_col == idx_b
        val = jnp.where(onehot, lane_io + 1, 0)
        wins = jnp.max(val, axis=1, keepdims=True)
        sel = ((val == wins) & (wins > 0)).astype(jnp.float32)
        gf_b = sel @ comb[b * M:(b + 1) * M]
        gf_ref[b, :, :] = gf_b
        gfs.append(gf_b)
    gf = jnp.concatenate(gfs, axis=0)

    hd = jnp.maximum(matT(gf, dnw1t_ref) + rowvec(dnb1_ref), 0.0)
    defect = hd @ dnw2_ref[...] + rowvec(dnb2_ref)

    proc_ref[...] = jnp.transpose(defect.reshape(B, G, 64), (0, 2, 1))

    hc = jnp.maximum(matT(defect, clw1t_ref) + rowvec(clb1_ref), 0.0)
    logits = matT(hc, clw2t_ref) + rowvec(clb2_ref)
    probs = jax.nn.softmax(logits, axis=-1)
    probs3 = jnp.transpose(probs.reshape(B, M, 5), (0, 2, 1))
    for b in range(B):
        probs_ref[:, b, :] = probs3[b]

    hs = jnp.maximum(matT(defect, svw1t_ref) + rowvec(svb1_ref), 0.0)
    sev_pre = jnp.sum(hs * rowvec(svw2t_ref), axis=1, keepdims=True)
    sev = jax.nn.sigmoid(sev_pre + svb2_ref[...])
    sev3 = jnp.transpose(sev.reshape(B, M, 1), (0, 2, 1))
    for b in range(B):
        sev_ref[b:b + 1, :] = sev3[b]


def kernel(point_cloud, normals, colors, grid_points, params):
    t = jnp.transpose
    pts_t = t(point_cloud, (2, 0, 1))
    nrm_t = t(normals, (2, 0, 1))
    col_t = t(colors, (2, 0, 1))

    p = params
    weight_args = [
        p["pt_W1"], p["pt_b1"], p["pt_W2"], p["pt_b2"], p["pt_W3"], p["pt_b3"],
        p["nm_W1"], p["nm_b1"], p["nm_W2"], p["nm_b2"], p["nm_W3"], p["nm_b3"],
        p["tx_W1"], p["tx_b1"], p["tx_W2"], p["tx_b2"], p["tx_W3"], p["tx_b3"],
        t(p["dn_W1"]), p["dn_b1"], p["dn_W2"], p["dn_b2"],
        t(p["cl_W1"]), p["cl_b1"], t(p["cl_W2"]), p["cl_b2"],
        t(p["sv_W1"]), p["sv_b1"], t(p["sv_W2"]), p["sv_b2"].reshape(1, 1),
    ]

    out_shapes = (
        jax.ShapeDtypeStruct((5, B, G), jnp.float32),
        jax.ShapeDtypeStruct((B, G), jnp.float32),
        jax.ShapeDtypeStruct((B, 64, G), jnp.float32),
        jax.ShapeDtypeStruct((B, G, F), jnp.float32),
    )

    first64 = pl.BlockSpec((3, B, 128), lambda i: (0, 0, 0))
    full = lambda a: pl.BlockSpec(a.shape, lambda i: (0,) * a.ndim)
    gpt = t(grid_points)

    probs5, sev, proc, gf = pl.pallas_call(
        _qc_kernel,
        out_shape=out_shapes,
        grid=(1,),
        in_specs=[first64, first64, first64, full(gpt)]
        + [full(w) for w in weight_args],
        out_specs=tuple(
            pl.BlockSpec(s.shape, lambda i, n=len(s.shape): (0,) * n)
            for s in out_shapes),
    )(pts_t, nrm_t, col_t, gpt, *weight_args)

    return (jnp.transpose(probs5, (1, 2, 0)), sev, proc,
            jnp.transpose(gf, (0, 2, 1)))
